# Initial kernel scaffold; baseline (speedup 1.0000x reference)
import jax
import jax.numpy as jnp
from jax.experimental import pallas as pl


def kernel(tokens, table, W1, b1, W2, b2):
    raise NotImplementedError("write your pallas kernel here")


# speedup vs baseline: 1.0571x; 1.0571x over previous
import math

import jax
import jax.numpy as jnp
from jax import lax
from jax.experimental import pallas as pl
from jax.experimental.pallas import tpu as pltpu
from jax.experimental.pallas import tpu_sc as plsc

B, L, V, D, O = 4096, 200, 1000000, 64, 32
NC, NS = 2, 16
NW = NC * NS
BPW = B // NW
C0 = 128
C1 = L - C0
ROW_UNROLL = 8


def _sc_gather_sum(tokens, table):
  mesh = plsc.VectorSubcoreMesh(core_axis_name="c", subcore_axis_name="s")

  def body(tokens_hbm, table_hbm, out_hbm, idx_v, buf_a, buf_b, out_v,
           sem_a, sem_b):
    wid = lax.axis_index("s") * NC + lax.axis_index("c")
    base = wid * BPW
    pltpu.sync_copy(tokens_hbm.at[pl.ds(base, BPW)], idx_v)

    def issue(bag, buf, sem):
      pltpu.async_copy(table_hbm.at[idx_v.at[bag, pl.ds(0, C0)]],
                       buf.at[pl.ds(0, C0)], sem)
      pltpu.async_copy(table_hbm.at[idx_v.at[bag, pl.ds(C0, C1)]],
                       buf.at[pl.ds(C0, C1)], sem)

    def wait(buf, sem):
      pltpu.make_async_copy(table_hbm.at[pl.ds(0, L)], buf, sem).wait()

    def accumulate(bag, buf):
      zeros = jnp.zeros((16,), jnp.float32)

      def step(i, accs):
        r = i * ROW_UNROLL
        new = list(accs)
        for dr in range(ROW_UNROLL):
          for j in range(4):
            new[j] = new[j] + buf[r + dr, pl.ds(16 * j, 16)]
        return tuple(new)

      accs = lax.fori_loop(0, L // ROW_UNROLL, step,
                           (zeros, zeros, zeros, zeros))
      for j in range(4):
        out_v[bag, pl.ds(16 * j, 16)] = accs[j]

    issue(0, buf_a, sem_a)

    def pair(p, carry):
      bag = p * 2
      issue(bag + 1, buf_b, sem_b)
      wait(buf_a, sem_a)
      accumulate(bag, buf_a)

      @pl.when(bag + 2 < BPW)
      def _():
        issue(bag + 2, buf_a, sem_a)

      wait(buf_b, sem_b)
      accumulate(bag + 1, buf_b)
      return carry

    lax.fori_loop(0, BPW // 2, pair, 0)
    pltpu.sync_copy(out_v, out_hbm.at[pl.ds(base, BPW)])

  return pl.kernel(
      body,
      out_type=jax.ShapeDtypeStruct((B, D), jnp.float32),
      mesh=mesh,
      scratch_types=[
          pltpu.VMEM((BPW, L), jnp.int32),
          pltpu.VMEM((L, D), jnp.float32),
          pltpu.VMEM((L, D), jnp.float32),
          pltpu.VMEM((BPW, D), jnp.float32),
          pltpu.SemaphoreType.DMA,
          pltpu.SemaphoreType.DMA,
      ],
      compiler_params=pltpu.CompilerParams(use_tc_tiling_on_sc=False),
  )(tokens, table)


def _tc_head(tokens, sums, W1, b1, W2, b2):
  def body(tok_ref, sums_ref, w1_ref, b1_ref, w2_ref, b2_ref, out_ref):
    t = tok_ref[...]
    cnt = jnp.sum((t != 0).astype(jnp.float32), axis=1, keepdims=True)
    pooled = sums_ref[...] / jnp.maximum(cnt, 1.0)
    h = jnp.dot(pooled, w1_ref[...],
                preferred_element_type=jnp.float32) + b1_ref[...]
    h = 0.5 * h * (1.0 + lax.erf(h * (1.0 / math.sqrt(2.0))))
    out_ref[...] = jnp.dot(h, w2_ref[...],
                           preferred_element_type=jnp.float32) + b2_ref[...]

  grid = 8
  bb = B // grid
  return pl.pallas_call(
      body,
      out_shape=jax.ShapeDtypeStruct((B, O), jnp.float32),
      grid=(grid,),
      in_specs=[
          pl.BlockSpec((bb, L), lambda i: (i, 0)),
          pl.BlockSpec((bb, D), lambda i: (i, 0)),
          pl.BlockSpec((D, D), lambda i: (0, 0)),
          pl.BlockSpec((1, D), lambda i: (0, 0)),
          pl.BlockSpec((D, O), lambda i: (0, 0)),
          pl.BlockSpec((1, O), lambda i: (0, 0)),
      ],
      out_specs=pl.BlockSpec((bb, O), lambda i: (i, 0)),
  )(tokens, sums, W1, b1, W2, b2)


def kernel(tokens, table, W1, b1, W2, b2):
  tokens = tokens.astype(jnp.int32)
  sums = _sc_gather_sum(tokens, table)
  return _tc_head(tokens, sums, W1, b1.reshape(1, D), W2, b2.reshape(1, O))
